# baseline (speedup 1.0000x reference)
import functools

import jax
import jax.numpy as jnp
from jax import lax
from jax.experimental import pallas as pl
from jax.experimental.pallas import tpu as pltpu
from jax.experimental.pallas import tpu_sc as plsc

L = 16


def _sc_pre_act(x, packed_chunks, row_lo, rows):
    B, N = x.shape
    NCH, K, HC = packed_chunks.shape
    H = NCH * HC
    NW = 16
    rpw = rows // NW

    mesh = plsc.VectorSubcoreMesh(
        core_axis_name="c", subcore_axis_name="s", num_cores=1
    )

    @functools.partial(
        pl.kernel,
        mesh=mesh,
        out_type=jax.ShapeDtypeStruct((rows, H), jnp.float32),
        compiler_params=pltpu.CompilerParams(
            use_tc_tiling_on_sc=True, needs_layout_passes=False
        ),
        scratch_types=[
            pltpu.VMEM((N,), jnp.float32),
            pltpu.VMEM((2, K, HC), jnp.uint32),
            pltpu.VMEM((H,), jnp.float32),
            pltpu.VMEM_SHARED((NCH, K, HC), jnp.uint32),
            pltpu.SemaphoreType.DMA,
            pltpu.SemaphoreType.DMA,
        ],
    )
    def sc_fn(x_hbm, pk_hbm, out_hbm, xrow, pkv, accv, pk_sp, s0, s1):
        wid = lax.axis_index("s")
        sems = (s0, s1)

        @pl.when(lax.axis_index("s") == 0)
        def _():
            pltpu.sync_copy(pk_hbm, pk_sp)

        plsc.subcore_barrier()

        def start_fetch(c, parity):
            pltpu.async_copy(pk_sp.at[c % NCH], pkv.at[parity], sems[parity])

        def wait_fetch(parity):
            pltpu.make_async_copy(pk_sp.at[0], pkv.at[parity], sems[parity]).wait()

        start_fetch(0, 0)
        start_fetch(1, 1)

        def do_chunk(c, parity):
            wait_fetch(parity)

            @plsc.parallel_loop(0, HC // L, unroll=2)
            def blk_body(j):
                accs = [jnp.zeros((L,), jnp.float32) for _ in range(4)]
                for half in range(2):
                    ks = range(half * 8, half * 8 + 8)
                    pvs = [pkv[parity, k, pl.ds(j * L, L)] for k in ks]
                    ivs = [
                        plsc.bitcast(jnp.right_shift(pv, jnp.uint32(15)), jnp.int32)
                        for pv in pvs
                    ]
                    wvs = [
                        plsc.bitcast(jnp.left_shift(pv, jnp.uint32(17)), jnp.float32)
                        for pv in pvs
                    ]
                    gs = [plsc.load_gather(xrow, [iv]) for iv in ivs]
                    for t in range(8):
                        accs[t % 4] = accs[t % 4] + gs[t] * wvs[t]
                acc = (accs[0] + accs[1]) + (accs[2] + accs[3])
                accv[pl.ds(c * HC + j * L, L)] = acc

            start_fetch(c + 2, parity)

        def row_body(i, carry):
            r = wid * rpw + i
            pltpu.sync_copy(x_hbm.at[row_lo + r], xrow)

            def chunk_pair(p, carry2):
                c = p * 2
                do_chunk(c, 0)
                do_chunk(c + 1, 1)
                return carry2

            lax.fori_loop(0, NCH // 2, chunk_pair, 0)
            pltpu.sync_copy(accv, out_hbm.at[r])
            return carry

        lax.fori_loop(0, rpw, row_body, 0)
        wait_fetch(0)
        wait_fetch(1)

    return sc_fn(x, packed_chunks)


def _tc_head(pre, b1, w2, b2):
    B, H = pre.shape
    BT = 256

    w2_pad = jnp.zeros((128, H), jnp.float32).at[0].set(w2[0])

    def body(b2_ref, pre_ref, b1_ref, w2_ref, out_ref):
        z = jnp.tanh(pre_ref[...] + b1_ref[...])
        s = lax.dot_general(z, w2_ref[...], (((1,), (1,)), ((), ())))
        out_ref[...] = jnp.tanh(s + b2_ref[0])

    out = pl.pallas_call(
        body,
        grid=(B // BT,),
        in_specs=[
            pl.BlockSpec(memory_space=pltpu.SMEM),
            pl.BlockSpec((BT, H), lambda i: (i, 0)),
            pl.BlockSpec((1, H), lambda i: (0, 0)),
            pl.BlockSpec((128, H), lambda i: (0, 0)),
        ],
        out_specs=pl.BlockSpec((BT, 128), lambda i: (i, 0)),
        out_shape=jax.ShapeDtypeStruct((B, 128), jnp.float32),
    )(b2, pre, b1.reshape(1, H), w2_pad)
    return out[:, 0]


def _pack_wiring(input_indices, w1, NCH, HC):
    K = input_indices.shape[1]
    wbits = lax.bitcast_convert_type(w1, jnp.uint32)
    wtop = jnp.right_shift(wbits + jnp.uint32(1 << 16), jnp.uint32(17))
    packed = jnp.left_shift(input_indices.astype(jnp.uint32), jnp.uint32(15)) | wtop
    return packed.T.reshape(K, NCH, HC).transpose(1, 0, 2)


def kernel(x, input_indices, w1, b1, w2, b2):
    H, K = input_indices.shape
    HC = 512
    NCH = H // HC
    packed_chunks = _pack_wiring(input_indices, w1, NCH, HC)
    B = x.shape[0]
    pre0 = _sc_pre_act(x, packed_chunks, 0, B // 2)
    pre1 = _sc_pre_act(x, packed_chunks, B // 2, B // 2)
    pre = jnp.concatenate([pre0, pre1], axis=0)
    return _tc_head(pre, b1, w2, b2)


# speedup vs baseline: 1.3835x; 1.3835x over previous
import functools

import jax
import jax.numpy as jnp
from jax import lax
from jax.experimental import pallas as pl
from jax.experimental.pallas import tpu as pltpu
from jax.experimental.pallas import tpu_sc as plsc

L = 16


def _sc_pre_act(x, packed_chunks):
    B, N = x.shape
    NCH, K, HC = packed_chunks.shape
    H = NCH * HC
    NW = 32
    rpw = B // NW

    mesh = plsc.VectorSubcoreMesh(core_axis_name="c", subcore_axis_name="s")

    @functools.partial(
        pl.kernel,
        mesh=mesh,
        out_type=jax.ShapeDtypeStruct((B, H), jnp.float32),
        compiler_params=pltpu.CompilerParams(
            use_tc_tiling_on_sc=True, needs_layout_passes=False
        ),
        scratch_types=[
            pltpu.VMEM((N,), jnp.float32),
            pltpu.VMEM((2, K, HC), jnp.uint32),
            pltpu.VMEM((2, H), jnp.float32),
            pltpu.VMEM_SHARED((NCH, K, HC), jnp.uint32),
            pltpu.SemaphoreType.DMA,
            pltpu.SemaphoreType.DMA,
            pltpu.SemaphoreType.DMA,
            pltpu.SemaphoreType.DMA,
        ],
    )
    def sc_fn(x_hbm, pk_hbm, out_hbm, xrow, pkv, accv, pk_sp, s0, s1, o0, o1):
        wid = lax.axis_index("s") * 2 + lax.axis_index("c")
        sems = (s0, s1)
        osems = (o0, o1)

        @pl.when(lax.axis_index("s") == 0)
        def _():
            pltpu.sync_copy(pk_hbm, pk_sp)

        plsc.subcore_barrier()

        def start_fetch(c, parity):
            pltpu.async_copy(pk_sp.at[c % NCH], pkv.at[parity], sems[parity])

        def wait_fetch(parity):
            pltpu.make_async_copy(pk_sp.at[0], pkv.at[parity], sems[parity]).wait()

        start_fetch(0, 0)
        start_fetch(1, 1)

        def do_chunk(c, parity, oslot):
            wait_fetch(parity)

            @plsc.parallel_loop(0, HC // L, unroll=2)
            def blk_body(j):
                accs = [jnp.zeros((L,), jnp.float32) for _ in range(4)]
                for half in range(2):
                    ks = range(half * 8, half * 8 + 8)
                    pvs = [pkv[parity, k, pl.ds(j * L, L)] for k in ks]
                    ivs = [
                        plsc.bitcast(jnp.right_shift(pv, jnp.uint32(15)), jnp.int32)
                        for pv in pvs
                    ]
                    wvs = [
                        plsc.bitcast(jnp.left_shift(pv, jnp.uint32(17)), jnp.float32)
                        for pv in pvs
                    ]
                    gs = [plsc.load_gather(xrow, [iv]) for iv in ivs]
                    for t in range(8):
                        accs[t % 4] = accs[t % 4] + gs[t] * wvs[t]
                acc = (accs[0] + accs[1]) + (accs[2] + accs[3])
                accv[oslot, pl.ds(c * HC + j * L, L)] = acc

            start_fetch(c + 2, parity)

        def do_row(i, oslot):
            r = wid * rpw + i
            pltpu.sync_copy(x_hbm.at[r], xrow)

            @pl.when(i >= 2)
            def _():
                pltpu.make_async_copy(
                    accv.at[oslot], out_hbm.at[r], osems[oslot]
                ).wait()

            def chunk_pair(p, carry2):
                c = p * 2
                do_chunk(c, 0, oslot)
                do_chunk(c + 1, 1, oslot)
                return carry2

            lax.fori_loop(0, NCH // 2, chunk_pair, 0)
            pltpu.async_copy(accv.at[oslot], out_hbm.at[r], osems[oslot])

        def row_pair(i2, carry):
            do_row(i2 * 2, 0)
            do_row(i2 * 2 + 1, 1)
            return carry

        lax.fori_loop(0, rpw // 2, row_pair, 0)
        pltpu.make_async_copy(accv.at[0], out_hbm.at[0], osems[0]).wait()
        pltpu.make_async_copy(accv.at[1], out_hbm.at[0], osems[1]).wait()
        wait_fetch(0)
        wait_fetch(1)

    return sc_fn(x, packed_chunks)


def _tc_head(pre, b1, w2, b2):
    B, H = pre.shape
    BT = 256

    w2_pad = jnp.zeros((128, H), jnp.float32).at[0].set(w2[0])

    def body(b2_ref, pre_ref, b1_ref, w2_ref, out_ref):
        z = jnp.tanh(pre_ref[...] + b1_ref[...])
        s = lax.dot_general(z, w2_ref[...], (((1,), (1,)), ((), ())))
        out_ref[...] = jnp.tanh(s + b2_ref[0])

    out = pl.pallas_call(
        body,
        grid=(B // BT,),
        in_specs=[
            pl.BlockSpec(memory_space=pltpu.SMEM),
            pl.BlockSpec((BT, H), lambda i: (i, 0)),
            pl.BlockSpec((1, H), lambda i: (0, 0)),
            pl.BlockSpec((128, H), lambda i: (0, 0)),
        ],
        out_specs=pl.BlockSpec((BT, 128), lambda i: (i, 0)),
        out_shape=jax.ShapeDtypeStruct((B, 128), jnp.float32),
    )(b2, pre, b1.reshape(1, H), w2_pad)
    return out[:, 0]


def _pack_wiring(input_indices, w1, NCH, HC):
    K = input_indices.shape[1]
    wbits = lax.bitcast_convert_type(w1, jnp.uint32)
    wtop = jnp.right_shift(wbits + jnp.uint32(1 << 16), jnp.uint32(17))
    packed = jnp.left_shift(input_indices.astype(jnp.uint32), jnp.uint32(15)) | wtop
    return packed.T.reshape(K, NCH, HC).transpose(1, 0, 2)


def kernel(x, input_indices, w1, b1, w2, b2):
    H, K = input_indices.shape
    HC = 512
    NCH = H // HC
    packed_chunks = _pack_wiring(input_indices, w1, NCH, HC)
    pre = _sc_pre_act(x, packed_chunks)
    return _tc_head(pre, b1, w2, b2)
